# baseline (speedup 1.0000x reference)
import jax
import jax.numpy as jnp
from jax import lax
from jax.experimental import pallas as pl
from jax.experimental.pallas import tpu as pltpu
from jax.experimental.pallas import tpu_sc as plsc

NC = 2
NS = 16
NW = NC * NS
CH = 128
NBUF = 2
L = 16
DUNROLL = 8


def _transpose_chunk(buf, tbuf, D):
  iota = lax.iota(jnp.int32, L)
  lidx = [iota + L * m for m in range(CH // L)]
  zero = iota * 0

  @pl.loop(0, D, step=DUNROLL)
  def _td(d0):
    for dd in range(DUNROLL):
      d = d0 + dd
      didx = zero + d
      for m in range(CH // L):
        v = plsc.load_gather(buf, [lidx[m], didx])
        tbuf[pl.ds(d * CH + L * m, L)] = v


def _make_body(B, H, D):
  ncn = B // (NW * CH)
  nce = (B * H) // (NW * CH)
  assert nce % NBUF == 0
  tpd = D // 8
  cph = B // CH

  def body(nodes_hbm, neigh_hbm, user_hbm, item_hbm,
           node_out, neigh_out, idx_n_v, idx_e_v, *scratch):
    bufs = scratch[:NBUF]
    tbufs = scratch[NBUF:2 * NBUF]
    gsem = scratch[2 * NBUF:3 * NBUF]
    ssem = scratch[3 * NBUF:4 * NBUF]

    w = lax.axis_index("s") * NC + lax.axis_index("c")
    pltpu.sync_copy(nodes_hbm.at[w], idx_n_v)
    pltpu.sync_copy(neigh_hbm.at[w], idx_e_v)

    def store_chunk(tbuf, out, g, sem):
      mbase = ((g // cph) * tpd) * cph + (g % cph)
      for i in range(tpd):
        pltpu.async_copy(tbuf.at[pl.ds(i * 1024, 1024)],
                         out.at[pl.ds((mbase + i * cph) * 1024, 1024)], sem)

    def wait_store(tbuf, out, g, sem):
      mbase = ((g // cph) * tpd) * cph + (g % cph)
      for i in range(tpd):
        pltpu.make_async_copy(
            tbuf.at[pl.ds(i * 1024, 1024)],
            out.at[pl.ds((mbase + i * cph) * 1024, 1024)], sem).wait()

    for j in range(ncn):
      b = j % NBUF
      if j >= NBUF:
        wait_store(tbufs[b], node_out, w * ncn + (j - NBUF), ssem[b])
      pltpu.async_copy(user_hbm.at[idx_n_v.at[j]], bufs[b], gsem[b])
      pltpu.make_async_copy(user_hbm.at[idx_n_v.at[j]], bufs[b],
                            gsem[b]).wait()
      _transpose_chunk(bufs[b], tbufs[b], D)
      store_chunk(tbufs[b], node_out, w * ncn + j, ssem[b])
    for j in range(max(0, ncn - NBUF), ncn):
      b = j % NBUF
      wait_store(tbufs[b], node_out, w * ncn + j, ssem[b])

    for b in range(NBUF):
      pltpu.async_copy(item_hbm.at[idx_e_v.at[b]], bufs[b], gsem[b])

    @pl.loop(0, nce, step=NBUF)
    def _round(c0):
      for b in range(NBUF):
        c = c0 + b
        pltpu.make_async_copy(item_hbm.at[idx_e_v.at[c]], bufs[b],
                              gsem[b]).wait()
        _transpose_chunk(bufs[b], tbufs[b], D)
        store_chunk(tbufs[b], neigh_out, w * nce + c, ssem[b])
      for b in range(NBUF):
        c = c0 + b
        wait_store(tbufs[b], neigh_out, w * nce + c, ssem[b])
        cn = lax.rem(c + NBUF, nce)
        pltpu.async_copy(item_hbm.at[idx_e_v.at[cn]], bufs[b], gsem[b])

    for b in range(NBUF):
      pltpu.make_async_copy(item_hbm.at[idx_e_v.at[b]], bufs[b],
                            gsem[b]).wait()

  return body, ncn, nce


def kernel(nodes, neighbors, degrees, user_table, item_table):
  B, H = neighbors.shape
  D = user_table.shape[1]
  assert B % (NW * CH) == 0 and (B * H) % (NW * CH) == 0 and D % 8 == 0

  body, ncn, nce = _make_body(B, H, D)
  tpd = D // 8

  mesh = plsc.VectorSubcoreMesh(
      core_axis_name="c", subcore_axis_name="s",
      num_cores=NC, num_subcores=NS)

  scratch = ([pltpu.VMEM((ncn, CH), jnp.int32),
              pltpu.VMEM((nce, CH), jnp.int32)]
             + [pltpu.VMEM((CH, D), jnp.float32) for _ in range(NBUF)]
             + [pltpu.VMEM((CH * D,), jnp.float32) for _ in range(NBUF)]
             + [pltpu.SemaphoreType.DMA for _ in range(2 * NBUF)])

  run = pl.kernel(
      body,
      out_type=(
          jax.ShapeDtypeStruct((B * D,), user_table.dtype),
          jax.ShapeDtypeStruct((B * H * D,), item_table.dtype),
      ),
      mesh=mesh,
      compiler_params=pltpu.CompilerParams(
          use_tc_tiling_on_sc=False, needs_layout_passes=False),
      scratch_types=scratch,
  )

  nodes_r = nodes.astype(jnp.int32).reshape(NW, ncn, CH)
  neigh_r = neighbors.astype(jnp.int32).T.reshape(NW, nce, CH)
  node_flat, neigh_flat = run(nodes_r, neigh_r, user_table, item_table)

  node_emb = (node_flat.reshape(tpd, B // CH, 8, CH)
              .transpose(1, 3, 0, 2).reshape(B, D))
  neigh_emb = (neigh_flat.reshape(H, tpd, B // CH, 8, CH)
               .transpose(2, 4, 0, 1, 3).reshape(B, H, D))
  return (node_emb, neigh_emb, degrees)
